# baseline (speedup 1.0000x reference)
import functools
import math

import jax
import jax.numpy as jnp
from jax import lax
from jax.experimental import pallas as pl
from jax.experimental.pallas import tpu as pltpu
from jax.experimental.pallas import tpu_sc as plsc

D_MODEL = 64
SCALE = math.sqrt(D_MODEL)
CHUNK = 128
NBUF = 2


_RBLK = 1024


def _repack_body(t_ref, out_ref):
    t = t_ref[...].T
    pieces = []
    for qq in range(_RBLK // 128):
        pieces.append(jnp.concatenate(
            [t[qq * 128:qq * 128 + 64], t[qq * 128 + 64:qq * 128 + 128]],
            axis=1))
    out_ref[...] = jnp.concatenate(pieces, axis=0)


def _packed_rows(V):
    return 64 * ((V + 127) // 128)


def _repack(tt, V):
    grid = (V + _RBLK - 1) // _RBLK
    return pl.pallas_call(
        _repack_body,
        grid=(grid,),
        in_specs=[pl.BlockSpec((D_MODEL, _RBLK), lambda i: (0, i))],
        out_specs=pl.BlockSpec((_RBLK // 2, 2 * D_MODEL), lambda i: (i, 0)),
        out_shape=jax.ShapeDtypeStruct((_packed_rows(V), 2 * D_MODEL),
                                       jnp.float32),
    )(tt)


@functools.lru_cache(maxsize=None)
def _build(B, V, n_rows, n_cols):
    info = plsc.get_sparse_core_info()
    NW = info.num_cores * info.num_subcores
    NC = info.num_cores
    assert B % (NW * CHUNK) == 0 and V % 2 == 0
    b_per_w = B // NW
    n_chunks = b_per_w // CHUNK
    assert n_chunks % NBUF == 0

    mesh = plsc.VectorSubcoreMesh(core_axis_name="c", subcore_axis_name="s")

    scratch = [pltpu.VMEM((b_per_w,), jnp.int32)]
    scratch += [pltpu.VMEM((CHUNK, 2 * D_MODEL), jnp.float32) for _ in range(NBUF)]
    scratch += [pltpu.VMEM((CHUNK // 2, 2 * D_MODEL), jnp.float32) for _ in range(NBUF)]
    scratch += [pltpu.VMEM((CHUNK,), jnp.int32) for _ in range(NBUF)]
    scratch += [pltpu.SemaphoreType.DMA for _ in range(3 * NBUF)]

    @functools.partial(
        pl.kernel,
        mesh=mesh,
        compiler_params=pltpu.CompilerParams(use_tc_tiling_on_sc=True),
        out_type=jax.ShapeDtypeStruct((B // 2, 2 * D_MODEL), jnp.float32),
        scratch_types=scratch,
    )
    def emb_kernel(x_hbm, q_hbm, table_hbm, out_hbm, qv, *rest):
        gbufs = rest[:NBUF]
        sbufs = rest[NBUF:2 * NBUF]
        parv = rest[2 * NBUF:3 * NBUF]
        sem_g = rest[3 * NBUF:4 * NBUF]
        sem_p = rest[4 * NBUF:5 * NBUF]
        sem_s = rest[5 * NBUF:6 * NBUF]
        wid = lax.axis_index("s") * NC + lax.axis_index("c")
        base = wid * b_per_w

        pltpu.sync_copy(
            q_hbm.at[pl.ds(pl.multiple_of(base, CHUNK), b_per_w)], qv)

        def cds(c, size):
            return pl.ds(pl.multiple_of(c * CHUNK, CHUNK), size)

        def gather_start(c, b):
            pltpu.make_async_copy(
                table_hbm.at[qv.at[cds(c, CHUNK)]], gbufs[b], sem_g[b]).start()
            pltpu.make_async_copy(
                x_hbm.at[pl.ds(pl.multiple_of(base + c * CHUNK, CHUNK), CHUNK)],
                parv[b], sem_p[b]).start()

        def gather_wait(b):
            pltpu.make_async_copy(
                table_hbm.at[qv.at[cds(0, CHUNK)]], gbufs[b], sem_g[b]).wait()
            pltpu.make_async_copy(
                x_hbm.at[pl.ds(0, CHUNK)], parv[b], sem_p[b]).wait()

        def store_start(c, b):
            pltpu.make_async_copy(
                sbufs[b],
                out_hbm.at[pl.ds(
                    pl.multiple_of((base + c * CHUNK) // 2, CHUNK // 2),
                    CHUNK // 2)],
                sem_s[b]).start()

        def store_wait(b):
            pltpu.make_async_copy(
                sbufs[b], out_hbm.at[pl.ds(0, CHUNK // 2)], sem_s[b]).wait()

        for b in range(NBUF - 1):
            gather_start(b, b)

        def step(c, b):
            gather_wait(b)
            G = gbufs[b]
            S = sbufs[b]
            P = parv[b]

            def group_body(g, _):
                pv = ((P[pl.ds(g * 16, 16)] >> 6) & 1) * D_MODEL
                for j in range(16):
                    r = g * 16 + j
                    s = g * 8 + j // 2
                    p = pv[j]
                    col0 = (j % 2) * D_MODEL
                    for k in range(D_MODEL // 16):
                        S[s, pl.ds(col0 + k * 16, 16)] = (
                            G[r, pl.ds(p + k * 16, 16)] * SCALE)
                return 0

            lax.fori_loop(0, CHUNK // 16, group_body, 0)
            store_start(c, b)
            bp = (b - 1) % NBUF
            p = c + NBUF - 1

            @pl.when((c >= 1) & (p < n_chunks))
            def _():
                store_wait(bp)

            @pl.when(p < n_chunks)
            def _():
                gather_start(p, bp)

        def outer_body(o, _):
            for b in range(NBUF):
                step(o * NBUF + b, b)
            return 0

        lax.fori_loop(0, n_chunks // NBUF, outer_body, 0)

        for b in range(NBUF):
            store_wait(b)

    def run(x, table):
        xi = x.reshape(-1).astype(jnp.int32)
        table2 = _repack(table.T, V)
        qi = ((xi >> 7) << 6) | (xi & 63)
        out2 = emb_kernel(xi, qi, table2)
        return out2.reshape(n_rows, n_cols, D_MODEL)

    return run


def kernel(x, table):
    n_rows, n_cols = x.shape
    V = table.shape[0]
    return _build(n_rows * n_cols, V, n_rows, n_cols)(x, table)


# speedup vs baseline: 1.3304x; 1.3304x over previous
import functools
import math

import jax
import jax.numpy as jnp
from jax import lax
from jax.experimental import pallas as pl
from jax.experimental.pallas import tpu as pltpu
from jax.experimental.pallas import tpu_sc as plsc

D_MODEL = 64
SCALE = math.sqrt(D_MODEL)
CHUNK = 128
NBUF = 5


@functools.lru_cache(maxsize=None)
def _build(B, V, n_rows, n_cols):
    info = plsc.get_sparse_core_info()
    NW = info.num_cores * info.num_subcores
    NC = info.num_cores
    assert B % (NW * CHUNK) == 0
    b_per_w = B // NW
    n_chunks = b_per_w // CHUNK
    assert n_chunks % NBUF == 0

    mesh = plsc.VectorSubcoreMesh(core_axis_name="c", subcore_axis_name="s")

    scratch = [pltpu.VMEM((n_chunks, CHUNK), jnp.int32)]
    scratch += [pltpu.VMEM((CHUNK, D_MODEL), jnp.float32) for _ in range(NBUF)]
    scratch += [pltpu.SemaphoreType.DMA for _ in range(2 * NBUF)]

    @functools.partial(
        pl.kernel,
        mesh=mesh,
        compiler_params=pltpu.CompilerParams(use_tc_tiling_on_sc=False),
        out_type=jax.ShapeDtypeStruct((B, D_MODEL), jnp.float32),
        scratch_types=scratch,
    )
    def emb_kernel(idx_hbm, table_hbm, out_hbm, idx_v, *bufs_and_sems):
        bufs = bufs_and_sems[:NBUF]
        sem_g = bufs_and_sems[NBUF:2 * NBUF]
        sem_s = bufs_and_sems[2 * NBUF:]
        wid = lax.axis_index("s") * NC + lax.axis_index("c")
        base = wid * b_per_w

        pltpu.sync_copy(idx_hbm.at[wid], idx_v)

        def gather_start(c, b):
            pltpu.make_async_copy(
                table_hbm.at[idx_v.at[c]], bufs[b], sem_g[b]).start()

        def gather_wait(b):
            pltpu.make_async_copy(
                table_hbm.at[idx_v.at[0]], bufs[b], sem_g[b]).wait()

        def store_start(c, b):
            pltpu.make_async_copy(
                bufs[b], out_hbm.at[pl.ds(base + c * CHUNK, CHUNK)],
                sem_s[b]).start()

        def store_wait(b):
            pltpu.make_async_copy(
                bufs[b], out_hbm.at[pl.ds(base, CHUNK)], sem_s[b]).wait()

        for b in range(NBUF - 1):
            gather_start(b, b)

        def outer_body(o, _):
            for b in range(NBUF):
                c = o + b
                gather_wait(b)

                @plsc.parallel_loop(0, CHUNK, step=1, unroll=4)
                def scale_row(r):
                    for k in range(D_MODEL // 16):
                        sl = (r, pl.ds(k * 16, 16))
                        bufs[b][sl] = bufs[b][sl] * SCALE

                store_start(c, b)
                bp = (b - 1) % NBUF
                p = c + NBUF - 1

                @pl.when((c >= 1) & (p < n_chunks))
                def _():
                    store_wait(bp)

                @pl.when(p < n_chunks)
                def _():
                    gather_start(p, bp)

            return 0

        lax.fori_loop(0, n_chunks // NBUF, lambda i, _: outer_body(i * NBUF, 0), 0)

        for b in range(NBUF):
            store_wait(b)

    def run(x, table):
        idx3d = x.reshape(-1).astype(jnp.int32).reshape(NW, n_chunks, CHUNK)
        out = emb_kernel(idx3d, table)
        return out.reshape(n_rows, n_cols, D_MODEL)

    return run


def kernel(x, table):
    n_rows, n_cols = x.shape
    V = table.shape[0]
    return _build(n_rows * n_cols, V, n_rows, n_cols)(x, table)
